# Initial kernel scaffold; baseline (speedup 1.0000x reference)
import jax
import jax.numpy as jnp
from jax.experimental import pallas as pl


def kernel(x, edge_index, W_self0, W_neigh0, b0, W_self1, W_neigh1, b1, W_self2, W_neigh2, b2, ln_g0, ln_b0, ln_g1, ln_b1, W_cls, b_cls):
    raise NotImplementedError("write your pallas kernel here")


# speedup vs baseline: 3.2685x; 3.2685x over previous
import functools

import jax
import jax.numpy as jnp
from jax import lax
from jax.experimental import pallas as pl
from jax.experimental.pallas import tpu as pltpu
from jax.experimental.pallas import tpu_sc as plsc

N = 10000
E = 320000
D = 128
C = 64

NC = 2
NS = 16
NW = NC * NS
LANES = 128
CHUNKS = 80
PHASES = 2
PCHUNKS = CHUNKS // PHASES
EPW = CHUNKS * LANES
E_PAD = NW * EPW
N_AGG = 10240
N_DEG = 10240
DSLICE = N_DEG // NS


def _sc_segment_sum(h, src_r, dst_r, zrow):
    mesh = plsc.VectorSubcoreMesh(core_axis_name="c", subcore_axis_name="s")

    @functools.partial(
        pl.kernel,
        mesh=mesh,
        out_type=jax.ShapeDtypeStruct((NC * N, D), jnp.float32),
        scratch_types=[
            pltpu.VMEM((PCHUNKS, LANES), jnp.int32),
            pltpu.VMEM((PCHUNKS, LANES), jnp.int32),
            pltpu.VMEM((LANES, D), jnp.float32),
            pltpu.VMEM((LANES, D), jnp.float32),
            pltpu.VMEM_SHARED((N_AGG, D), jnp.float32),
            pltpu.SemaphoreType.DMA,
            pltpu.SemaphoreType.DMA,
        ],
    )
    def k(h_hbm, src_hbm, dst_hbm, z_hbm, out_hbm, src_v, dst_v, buf0, buf1,
          acc, sem0, sem1):
        c = lax.axis_index("c")
        s = lax.axis_index("s")
        wid = c * NS + s
        bufs = (buf0, buf1)
        sems = (sem0, sem1)

        pltpu.sync_copy(z_hbm, buf0)
        rows0 = N_AGG // NS
        for kk in range(rows0 // LANES):
            pltpu.sync_copy(buf0, acc.at[pl.ds(s * rows0 + kk * LANES, LANES)])
        plsc.subcore_barrier()

        def start_gather(j, b):
            pltpu.async_copy(h_hbm.at[src_v.at[j]], bufs[b], sems[b])

        for p in range(PHASES):
            pltpu.sync_copy(src_hbm.at[wid * PHASES + p], src_v)
            pltpu.sync_copy(dst_hbm.at[wid * PHASES + p], dst_v)
            start_gather(0, 0)
            start_gather(1, 1)

            def body(j0, carry):
                for b in range(2):
                    j = j0 * 2 + b
                    pltpu.make_async_copy(h_hbm.at[src_v.at[0]], bufs[b],
                                          sems[b]).wait()
                    pltpu.sync_copy(bufs[b], acc.at[dst_v.at[j]], add=True)

                    @pl.when(j + 2 < PCHUNKS)
                    def _():
                        start_gather(j + 2, b)

                return carry

            lax.fori_loop(0, PCHUNKS // 2, body, 0)
        plsc.subcore_barrier()

        full = N // LANES
        for i in range(pl.cdiv(full, NS)):
            m = s + NS * i

            @pl.when(m < full)
            def _():
                pltpu.sync_copy(acc.at[pl.ds(m * LANES, LANES)], buf0)
                pltpu.sync_copy(buf0, out_hbm.at[pl.ds(c * N + m * LANES, LANES)])

        tail = N - full * LANES

        @pl.when(s == NS - 1)
        def _():
            pltpu.sync_copy(acc.at[pl.ds(full * LANES, tail)],
                            buf0.at[pl.ds(0, tail)])
            pltpu.sync_copy(buf0.at[pl.ds(0, tail)],
                            out_hbm.at[pl.ds(c * N + full * LANES, tail)])

    return k(h, src_r, dst_r, zrow)


def _sc_degree(dst_r, ones_rows, zrow):
    mesh = plsc.VectorSubcoreMesh(core_axis_name="c", subcore_axis_name="s")

    @functools.partial(
        pl.kernel,
        mesh=mesh,
        out_type=jax.ShapeDtypeStruct((NC * N_DEG, D), jnp.float32),
        scratch_types=[
            pltpu.VMEM((PCHUNKS, LANES), jnp.int32),
            pltpu.VMEM((LANES, D), jnp.float32),
            pltpu.VMEM_SHARED((N_DEG, D), jnp.float32),
        ],
    )
    def k(dst_hbm, ones_hbm, z_hbm, out_hbm, idx_v, ones_v, cnt):
        c = lax.axis_index("c")
        s = lax.axis_index("s")
        wid = c * NS + s
        pltpu.sync_copy(z_hbm, ones_v)
        for kk in range(DSLICE // LANES):
            pltpu.sync_copy(ones_v, cnt.at[pl.ds(s * DSLICE + kk * LANES, LANES)])
        pltpu.sync_copy(ones_hbm, ones_v)
        plsc.subcore_barrier()

        def body(j, carry):
            pltpu.sync_copy(ones_v, cnt.at[idx_v.at[j]], add=True)
            return carry

        for p in range(PHASES):
            pltpu.sync_copy(dst_hbm.at[wid * PHASES + p], idx_v)
            lax.fori_loop(0, PCHUNKS, body, 0)
        plsc.subcore_barrier()
        for kk in range(DSLICE // LANES):
            base = s * DSLICE + kk * LANES
            pltpu.sync_copy(cnt.at[pl.ds(base, LANES)], ones_v)
            pltpu.sync_copy(ones_v, out_hbm.at[pl.ds(c * N_DEG + base, LANES)])

    return k(dst_r, ones_rows, zrow)


def _tc_layer(h, agg, deg, Ws, Wn, b, g, be):
    R = 2000

    def body(h_ref, a0, a1, d0, d1, ws, wn, bb, gg, bee, out_ref):
        hh = h_ref[...]
        inv = 1.0 / jnp.maximum(d0[0][:, 0:1] + d1[0][:, 0:1], 1.0)
        mean = (a0[0] + a1[0]) * inv
        y = jnp.dot(hh, ws[...], preferred_element_type=jnp.float32)
        y = y + jnp.dot(mean, wn[...], preferred_element_type=jnp.float32)
        y = y + bb[...] + hh
        y = jnp.maximum(y, 0.0)
        mu = jnp.mean(y, axis=1, keepdims=True)
        xc = y - mu
        var = jnp.mean(xc * xc, axis=1, keepdims=True)
        out_ref[...] = xc / jnp.sqrt(var + 1e-5) * gg[...] + bee[...]

    return pl.pallas_call(
        body,
        grid=(N // R,),
        in_specs=[
            pl.BlockSpec((R, D), lambda i: (i, 0)),
            pl.BlockSpec((1, R, D), lambda i: (0, i, 0)),
            pl.BlockSpec((1, R, D), lambda i: (1, i, 0)),
            pl.BlockSpec((1, R, D), lambda i: (0, i, 0)),
            pl.BlockSpec((1, R, D), lambda i: (1, i, 0)),
            pl.BlockSpec((D, D), lambda i: (0, 0)),
            pl.BlockSpec((D, D), lambda i: (0, 0)),
            pl.BlockSpec((1, D), lambda i: (0, 0)),
            pl.BlockSpec((1, D), lambda i: (0, 0)),
            pl.BlockSpec((1, D), lambda i: (0, 0)),
        ],
        out_specs=pl.BlockSpec((R, D), lambda i: (i, 0)),
        out_shape=jax.ShapeDtypeStruct((N, D), jnp.float32),
    )(h, agg, agg, deg, deg, Ws, Wn, b, g, be)


def _tc_final(h, agg, deg, Ws, Wn, b, Wc, bc):
    R = 2000

    def body(h_ref, a0, a1, d0, d1, ws, wn, bb, wc, bcc, lo_ref, emb_ref):
        hh = h_ref[...]
        inv = 1.0 / jnp.maximum(d0[0][:, 0:1] + d1[0][:, 0:1], 1.0)
        mean = (a0[0] + a1[0]) * inv
        y = jnp.dot(hh, ws[...], preferred_element_type=jnp.float32)
        y = y + jnp.dot(mean, wn[...], preferred_element_type=jnp.float32)
        y = y + bb[...] + hh
        emb_ref[...] = y
        lo_ref[...] = (jnp.dot(y, wc[...], preferred_element_type=jnp.float32)
                       + bcc[...])

    return pl.pallas_call(
        body,
        grid=(N // R,),
        in_specs=[
            pl.BlockSpec((R, D), lambda i: (i, 0)),
            pl.BlockSpec((1, R, D), lambda i: (0, i, 0)),
            pl.BlockSpec((1, R, D), lambda i: (1, i, 0)),
            pl.BlockSpec((1, R, D), lambda i: (0, i, 0)),
            pl.BlockSpec((1, R, D), lambda i: (1, i, 0)),
            pl.BlockSpec((D, D), lambda i: (0, 0)),
            pl.BlockSpec((D, D), lambda i: (0, 0)),
            pl.BlockSpec((1, D), lambda i: (0, 0)),
            pl.BlockSpec((D, C), lambda i: (0, 0)),
            pl.BlockSpec((1, C), lambda i: (0, 0)),
        ],
        out_specs=[
            pl.BlockSpec((R, C), lambda i: (i, 0)),
            pl.BlockSpec((R, D), lambda i: (i, 0)),
        ],
        out_shape=[
            jax.ShapeDtypeStruct((N, C), jnp.float32),
            jax.ShapeDtypeStruct((N, D), jnp.float32),
        ],
    )(h, agg, agg, deg, deg, Ws, Wn, b, Wc, bc)


def kernel(x, edge_index, W_self0, W_neigh0, b0, W_self1, W_neigh1, b1,
           W_self2, W_neigh2, b2, ln_g0, ln_b0, ln_g1, ln_b1, W_cls, b_cls):
    src = edge_index[0]
    dst = edge_index[1]
    pad = E_PAD - E
    src_r = jnp.concatenate(
        [src, jnp.zeros((pad,), jnp.int32)]).reshape(NW * PHASES, PCHUNKS, LANES)
    dst_r = jnp.concatenate(
        [dst, jnp.full((pad,), N, jnp.int32)]).reshape(NW * PHASES, PCHUNKS, LANES)
    zrow = jnp.zeros((LANES, D), jnp.float32)
    ones_rows = jnp.ones((LANES, D), jnp.float32)

    deg = _sc_degree(dst_r, ones_rows, zrow).reshape(NC, N_DEG, D)

    b0r, b1r, b2r = b0.reshape(1, D), b1.reshape(1, D), b2.reshape(1, D)
    g0r, be0r = ln_g0.reshape(1, D), ln_b0.reshape(1, D)
    g1r, be1r = ln_g1.reshape(1, D), ln_b1.reshape(1, D)
    bcr = b_cls.reshape(1, C)

    h = x
    agg = _sc_segment_sum(h, src_r, dst_r, zrow).reshape(NC, N, D)
    h = _tc_layer(h, agg, deg, W_self0, W_neigh0, b0r, g0r, be0r)
    agg = _sc_segment_sum(h, src_r, dst_r, zrow).reshape(NC, N, D)
    h = _tc_layer(h, agg, deg, W_self1, W_neigh1, b1r, g1r, be1r)
    agg = _sc_segment_sum(h, src_r, dst_r, zrow).reshape(NC, N, D)
    logits, emb = _tc_final(h, agg, deg, W_self2, W_neigh2, b2r, W_cls, bcr)
    return (logits, emb)


# speedup vs baseline: 3.3632x; 1.0290x over previous
import functools

import jax
import jax.numpy as jnp
from jax import lax
from jax.experimental import pallas as pl
from jax.experimental.pallas import tpu as pltpu
from jax.experimental.pallas import tpu_sc as plsc

N = 10000
E = 320000
D = 128
C = 64

NC = 2
NS = 16
NW = NC * NS
LANES = 128
CHUNKS = 80
PHASES = 2
PCHUNKS = CHUNKS // PHASES
GSUB = 4
GROWS = LANES // GSUB
EPW = CHUNKS * LANES
E_PAD = NW * EPW
N_AGG = 10240
N_DEG = 10240
DSLICE = N_DEG // NS


def _sc_segment_sum(h, src_r, dst_r, zrow):
    mesh = plsc.VectorSubcoreMesh(core_axis_name="c", subcore_axis_name="s")

    @functools.partial(
        pl.kernel,
        mesh=mesh,
        out_type=jax.ShapeDtypeStruct((NC * N, D), jnp.float32),
        scratch_types=[
            pltpu.VMEM((PCHUNKS, LANES), jnp.int32),
            pltpu.VMEM((PCHUNKS, LANES), jnp.int32),
            pltpu.VMEM((LANES, D), jnp.float32),
            pltpu.VMEM((LANES, D), jnp.float32),
            pltpu.VMEM_SHARED((N_AGG, D), jnp.float32),
            pltpu.SemaphoreType.DMA,
            pltpu.SemaphoreType.DMA,
        ],
    )
    def k(h_hbm, src_hbm, dst_hbm, z_hbm, out_hbm, src_v, dst_v, buf0, buf1,
          acc, sem0, sem1):
        c = lax.axis_index("c")
        s = lax.axis_index("s")
        wid = c * NS + s
        bufs = (buf0, buf1)
        sems = (sem0, sem1)

        pltpu.sync_copy(z_hbm, buf0)
        rows0 = N_AGG // NS
        for kk in range(rows0 // LANES):
            pltpu.sync_copy(buf0, acc.at[pl.ds(s * rows0 + kk * LANES, LANES)])
        plsc.subcore_barrier()

        def start_gather(j, b):
            for g in range(GSUB):
                pltpu.async_copy(h_hbm.at[src_v.at[j, pl.ds(g * GROWS, GROWS)]],
                                 bufs[b].at[pl.ds(g * GROWS, GROWS)], sems[b])

        def wait_gather(b):
            pltpu.make_async_copy(h_hbm.at[dst_v.at[0]], bufs[b],
                                  sems[b]).wait()

        for p in range(PHASES):
            pltpu.sync_copy(src_hbm.at[wid * PHASES + p], src_v)
            pltpu.sync_copy(dst_hbm.at[wid * PHASES + p], dst_v)
            start_gather(0, 0)
            start_gather(1, 1)

            def body(j0, carry):
                for b in range(2):
                    j = j0 * 2 + b
                    wait_gather(b)
                    pltpu.sync_copy(bufs[b], acc.at[dst_v.at[j]], add=True)

                    @pl.when(j + 2 < PCHUNKS)
                    def _():
                        start_gather(j + 2, b)

                return carry

            lax.fori_loop(0, PCHUNKS // 2, body, 0)
        plsc.subcore_barrier()

        full = N // LANES
        for i in range(pl.cdiv(full, NS)):
            m = s + NS * i

            @pl.when(m < full)
            def _():
                pltpu.sync_copy(acc.at[pl.ds(m * LANES, LANES)], buf0)
                pltpu.sync_copy(buf0, out_hbm.at[pl.ds(c * N + m * LANES, LANES)])

        tail = N - full * LANES

        @pl.when(s == NS - 1)
        def _():
            pltpu.sync_copy(acc.at[pl.ds(full * LANES, tail)],
                            buf0.at[pl.ds(0, tail)])
            pltpu.sync_copy(buf0.at[pl.ds(0, tail)],
                            out_hbm.at[pl.ds(c * N + full * LANES, tail)])

    return k(h, src_r, dst_r, zrow)


def _sc_degree(dst_r, ones_rows, zrow):
    mesh = plsc.VectorSubcoreMesh(core_axis_name="c", subcore_axis_name="s")

    @functools.partial(
        pl.kernel,
        mesh=mesh,
        out_type=jax.ShapeDtypeStruct((NC * N_DEG, D), jnp.float32),
        scratch_types=[
            pltpu.VMEM((PCHUNKS, LANES), jnp.int32),
            pltpu.VMEM((LANES, D), jnp.float32),
            pltpu.VMEM_SHARED((N_DEG, D), jnp.float32),
        ],
    )
    def k(dst_hbm, ones_hbm, z_hbm, out_hbm, idx_v, ones_v, cnt):
        c = lax.axis_index("c")
        s = lax.axis_index("s")
        wid = c * NS + s
        pltpu.sync_copy(z_hbm, ones_v)
        for kk in range(DSLICE // LANES):
            pltpu.sync_copy(ones_v, cnt.at[pl.ds(s * DSLICE + kk * LANES, LANES)])
        pltpu.sync_copy(ones_hbm, ones_v)
        plsc.subcore_barrier()

        def body(j, carry):
            pltpu.sync_copy(ones_v, cnt.at[idx_v.at[j]], add=True)
            return carry

        for p in range(PHASES):
            pltpu.sync_copy(dst_hbm.at[wid * PHASES + p], idx_v)
            lax.fori_loop(0, PCHUNKS, body, 0)
        plsc.subcore_barrier()
        for kk in range(DSLICE // LANES):
            base = s * DSLICE + kk * LANES
            pltpu.sync_copy(cnt.at[pl.ds(base, LANES)], ones_v)
            pltpu.sync_copy(ones_v, out_hbm.at[pl.ds(c * N_DEG + base, LANES)])

    return k(dst_r, ones_rows, zrow)


def _tc_layer(h, agg, deg, Ws, Wn, b, g, be):
    R = 2000

    def body(h_ref, a0, a1, d0, d1, ws, wn, bb, gg, bee, out_ref):
        hh = h_ref[...]
        inv = 1.0 / jnp.maximum(d0[0][:, 0:1] + d1[0][:, 0:1], 1.0)
        mean = (a0[0] + a1[0]) * inv
        y = jnp.dot(hh, ws[...], preferred_element_type=jnp.float32)
        y = y + jnp.dot(mean, wn[...], preferred_element_type=jnp.float32)
        y = y + bb[...] + hh
        y = jnp.maximum(y, 0.0)
        mu = jnp.mean(y, axis=1, keepdims=True)
        xc = y - mu
        var = jnp.mean(xc * xc, axis=1, keepdims=True)
        out_ref[...] = xc / jnp.sqrt(var + 1e-5) * gg[...] + bee[...]

    return pl.pallas_call(
        body,
        grid=(N // R,),
        in_specs=[
            pl.BlockSpec((R, D), lambda i: (i, 0)),
            pl.BlockSpec((1, R, D), lambda i: (0, i, 0)),
            pl.BlockSpec((1, R, D), lambda i: (1, i, 0)),
            pl.BlockSpec((1, R, D), lambda i: (0, i, 0)),
            pl.BlockSpec((1, R, D), lambda i: (1, i, 0)),
            pl.BlockSpec((D, D), lambda i: (0, 0)),
            pl.BlockSpec((D, D), lambda i: (0, 0)),
            pl.BlockSpec((1, D), lambda i: (0, 0)),
            pl.BlockSpec((1, D), lambda i: (0, 0)),
            pl.BlockSpec((1, D), lambda i: (0, 0)),
        ],
        out_specs=pl.BlockSpec((R, D), lambda i: (i, 0)),
        out_shape=jax.ShapeDtypeStruct((N, D), jnp.float32),
    )(h, agg, agg, deg, deg, Ws, Wn, b, g, be)


def _tc_final(h, agg, deg, Ws, Wn, b, Wc, bc):
    R = 2000

    def body(h_ref, a0, a1, d0, d1, ws, wn, bb, wc, bcc, lo_ref, emb_ref):
        hh = h_ref[...]
        inv = 1.0 / jnp.maximum(d0[0][:, 0:1] + d1[0][:, 0:1], 1.0)
        mean = (a0[0] + a1[0]) * inv
        y = jnp.dot(hh, ws[...], preferred_element_type=jnp.float32)
        y = y + jnp.dot(mean, wn[...], preferred_element_type=jnp.float32)
        y = y + bb[...] + hh
        emb_ref[...] = y
        lo_ref[...] = (jnp.dot(y, wc[...], preferred_element_type=jnp.float32)
                       + bcc[...])

    return pl.pallas_call(
        body,
        grid=(N // R,),
        in_specs=[
            pl.BlockSpec((R, D), lambda i: (i, 0)),
            pl.BlockSpec((1, R, D), lambda i: (0, i, 0)),
            pl.BlockSpec((1, R, D), lambda i: (1, i, 0)),
            pl.BlockSpec((1, R, D), lambda i: (0, i, 0)),
            pl.BlockSpec((1, R, D), lambda i: (1, i, 0)),
            pl.BlockSpec((D, D), lambda i: (0, 0)),
            pl.BlockSpec((D, D), lambda i: (0, 0)),
            pl.BlockSpec((1, D), lambda i: (0, 0)),
            pl.BlockSpec((D, C), lambda i: (0, 0)),
            pl.BlockSpec((1, C), lambda i: (0, 0)),
        ],
        out_specs=[
            pl.BlockSpec((R, C), lambda i: (i, 0)),
            pl.BlockSpec((R, D), lambda i: (i, 0)),
        ],
        out_shape=[
            jax.ShapeDtypeStruct((N, C), jnp.float32),
            jax.ShapeDtypeStruct((N, D), jnp.float32),
        ],
    )(h, agg, agg, deg, deg, Ws, Wn, b, Wc, bc)


def kernel(x, edge_index, W_self0, W_neigh0, b0, W_self1, W_neigh1, b1,
           W_self2, W_neigh2, b2, ln_g0, ln_b0, ln_g1, ln_b1, W_cls, b_cls):
    src = edge_index[0]
    dst = edge_index[1]
    pad = E_PAD - E
    src_r = jnp.concatenate(
        [src, jnp.zeros((pad,), jnp.int32)]).reshape(NW * PHASES, PCHUNKS, LANES)
    dst_r = jnp.concatenate(
        [dst, jnp.full((pad,), N, jnp.int32)]).reshape(NW * PHASES, PCHUNKS, LANES)
    zrow = jnp.zeros((LANES, D), jnp.float32)
    ones_rows = jnp.ones((LANES, D), jnp.float32)

    deg = _sc_degree(dst_r, ones_rows, zrow).reshape(NC, N_DEG, D)

    b0r, b1r, b2r = b0.reshape(1, D), b1.reshape(1, D), b2.reshape(1, D)
    g0r, be0r = ln_g0.reshape(1, D), ln_b0.reshape(1, D)
    g1r, be1r = ln_g1.reshape(1, D), ln_b1.reshape(1, D)
    bcr = b_cls.reshape(1, C)

    h = x
    agg = _sc_segment_sum(h, src_r, dst_r, zrow).reshape(NC, N, D)
    h = _tc_layer(h, agg, deg, W_self0, W_neigh0, b0r, g0r, be0r)
    agg = _sc_segment_sum(h, src_r, dst_r, zrow).reshape(NC, N, D)
    h = _tc_layer(h, agg, deg, W_self1, W_neigh1, b1r, g1r, be1r)
    agg = _sc_segment_sum(h, src_r, dst_r, zrow).reshape(NC, N, D)
    logits, emb = _tc_final(h, agg, deg, W_self2, W_neigh2, b2r, W_cls, bcr)
    return (logits, emb)


# speedup vs baseline: 6.8601x; 2.0398x over previous
import functools

import jax
import jax.numpy as jnp
from jax import lax
from jax.experimental import pallas as pl
from jax.experimental.pallas import tpu as pltpu
from jax.experimental.pallas import tpu_sc as plsc

N = 10000
E = 320000
D = 128
C = 64

NC = 2
NS = 16
NW = NC * NS
LANES = 128
CHUNKS = 80
PHASES = 2
PCHUNKS = CHUNKS // PHASES
GSUB = 4
GROWS = LANES // GSUB
EPW = CHUNKS * LANES
E_PAD = NW * EPW
N_AGG = 10240
N_DEG = 10240
DSLICE = N_DEG // NS


def _sc_expand(h, src_r):
    mesh = plsc.VectorSubcoreMesh(core_axis_name="c", subcore_axis_name="s")

    @functools.partial(
        pl.kernel,
        mesh=mesh,
        out_type=jax.ShapeDtypeStruct((E_PAD, D), jnp.float32),
        scratch_types=[
            pltpu.VMEM((PCHUNKS, LANES), jnp.int32),
            pltpu.VMEM((LANES, D), jnp.float32),
            pltpu.VMEM((LANES, D), jnp.float32),
            pltpu.VMEM_SHARED((N, D), jnp.float32),
            pltpu.SemaphoreType.DMA,
            pltpu.SemaphoreType.DMA,
            pltpu.SemaphoreType.DMA,
            pltpu.SemaphoreType.DMA,
        ],
    )
    def k(h_hbm, src_hbm, out_hbm, src_v, buf0, buf1, hst, gsem0, gsem1,
          wsem0, wsem1):
        c = lax.axis_index("c")
        s = lax.axis_index("s")
        wid = c * NS + s
        bufs = (buf0, buf1)
        gsems = (gsem0, gsem1)
        wsems = (wsem0, wsem1)

        full = N // LANES
        for i in range(pl.cdiv(full, NS)):
            m = s + NS * i

            @pl.when(m < full)
            def _():
                pltpu.sync_copy(h_hbm.at[pl.ds(m * LANES, LANES)], buf0)
                pltpu.sync_copy(buf0, hst.at[pl.ds(m * LANES, LANES)])

        tail = N - full * LANES

        @pl.when(s == NS - 1)
        def _():
            pltpu.sync_copy(h_hbm.at[pl.ds(full * LANES, tail)],
                            buf0.at[pl.ds(0, tail)])
            pltpu.sync_copy(buf0.at[pl.ds(0, tail)],
                            hst.at[pl.ds(full * LANES, tail)])

        plsc.subcore_barrier()

        def start_gather(j, b):
            for g in range(GSUB):
                pltpu.async_copy(hst.at[src_v.at[j, pl.ds(g * GROWS, GROWS)]],
                                 bufs[b].at[pl.ds(g * GROWS, GROWS)], gsems[b])

        def wait_gather(b):
            pltpu.make_async_copy(hst.at[src_v.at[0]], bufs[b],
                                  gsems[b]).wait()

        def start_write(base, b):
            pltpu.async_copy(bufs[b], out_hbm.at[pl.ds(base, LANES)], wsems[b])

        def wait_write(b):
            pltpu.make_async_copy(bufs[b], out_hbm.at[pl.ds(0, LANES)],
                                  wsems[b]).wait()

        for p in range(PHASES):
            pltpu.sync_copy(src_hbm.at[wid * PHASES + p], src_v)
            pbase = wid * EPW + p * PCHUNKS * LANES
            start_gather(0, 0)
            start_gather(1, 1)

            def body(j0, carry):
                for b in range(2):
                    j = j0 * 2 + b
                    wait_gather(b)
                    start_write(pbase + j * LANES, b)

                    @pl.when(j + 2 < PCHUNKS)
                    def _():
                        wait_write(b)
                        start_gather(j + 2, b)

                return carry

            lax.fori_loop(0, PCHUNKS // 2, body, 0)
            wait_write(0)
            wait_write(1)

    return k(h, src_r)


def _sc_collect(rows, dst_r, zrow):
    mesh = plsc.VectorSubcoreMesh(core_axis_name="c", subcore_axis_name="s")

    @functools.partial(
        pl.kernel,
        mesh=mesh,
        out_type=jax.ShapeDtypeStruct((NC * N, D), jnp.float32),
        scratch_types=[
            pltpu.VMEM((PCHUNKS, LANES), jnp.int32),
            pltpu.VMEM((LANES, D), jnp.float32),
            pltpu.VMEM((LANES, D), jnp.float32),
            pltpu.VMEM_SHARED((N_AGG, D), jnp.float32),
            pltpu.SemaphoreType.DMA,
            pltpu.SemaphoreType.DMA,
        ],
    )
    def k(rows_hbm, dst_hbm, z_hbm, out_hbm, dst_v, buf0, buf1, acc,
          sem0, sem1):
        c = lax.axis_index("c")
        s = lax.axis_index("s")
        wid = c * NS + s
        bufs = (buf0, buf1)
        sems = (sem0, sem1)

        pltpu.sync_copy(z_hbm, buf0)
        rows0 = N_AGG // NS
        for kk in range(rows0 // LANES):
            pltpu.sync_copy(buf0, acc.at[pl.ds(s * rows0 + kk * LANES, LANES)])
        plsc.subcore_barrier()

        def start_read(base, b):
            pltpu.async_copy(rows_hbm.at[pl.ds(base, LANES)], bufs[b], sems[b])

        def wait_read(b):
            pltpu.make_async_copy(rows_hbm.at[pl.ds(0, LANES)], bufs[b],
                                  sems[b]).wait()

        for p in range(PHASES):
            pltpu.sync_copy(dst_hbm.at[wid * PHASES + p], dst_v)
            pbase = wid * EPW + p * PCHUNKS * LANES
            start_read(pbase, 0)
            start_read(pbase + LANES, 1)

            def body(j0, carry):
                for b in range(2):
                    j = j0 * 2 + b
                    wait_read(b)
                    pltpu.sync_copy(bufs[b], acc.at[dst_v.at[j]], add=True)

                    @pl.when(j + 2 < PCHUNKS)
                    def _():
                        start_read(pbase + (j + 2) * LANES, b)

                return carry

            lax.fori_loop(0, PCHUNKS // 2, body, 0)
        plsc.subcore_barrier()

        full = N // LANES
        for i in range(pl.cdiv(full, NS)):
            m = s + NS * i

            @pl.when(m < full)
            def _():
                pltpu.sync_copy(acc.at[pl.ds(m * LANES, LANES)], buf0)
                pltpu.sync_copy(buf0, out_hbm.at[pl.ds(c * N + m * LANES, LANES)])

        tail = N - full * LANES

        @pl.when(s == NS - 1)
        def _():
            pltpu.sync_copy(acc.at[pl.ds(full * LANES, tail)],
                            buf0.at[pl.ds(0, tail)])
            pltpu.sync_copy(buf0.at[pl.ds(0, tail)],
                            out_hbm.at[pl.ds(c * N + full * LANES, tail)])

    return k(rows, dst_r, zrow)


def _sc_segment_sum(h, src_r, dst_r, zrow):
    rows = _sc_expand(h, src_r)
    return _sc_collect(rows, dst_r, zrow)


def _sc_degree(dst_r, ones_rows, zrow):
    mesh = plsc.VectorSubcoreMesh(core_axis_name="c", subcore_axis_name="s")

    @functools.partial(
        pl.kernel,
        mesh=mesh,
        out_type=jax.ShapeDtypeStruct((NC * N_DEG, D), jnp.float32),
        scratch_types=[
            pltpu.VMEM((PCHUNKS, LANES), jnp.int32),
            pltpu.VMEM((LANES, D), jnp.float32),
            pltpu.VMEM_SHARED((N_DEG, D), jnp.float32),
        ],
    )
    def k(dst_hbm, ones_hbm, z_hbm, out_hbm, idx_v, ones_v, cnt):
        c = lax.axis_index("c")
        s = lax.axis_index("s")
        wid = c * NS + s
        pltpu.sync_copy(z_hbm, ones_v)
        for kk in range(DSLICE // LANES):
            pltpu.sync_copy(ones_v, cnt.at[pl.ds(s * DSLICE + kk * LANES, LANES)])
        pltpu.sync_copy(ones_hbm, ones_v)
        plsc.subcore_barrier()

        def body(j, carry):
            pltpu.sync_copy(ones_v, cnt.at[idx_v.at[j]], add=True)
            return carry

        for p in range(PHASES):
            pltpu.sync_copy(dst_hbm.at[wid * PHASES + p], idx_v)
            lax.fori_loop(0, PCHUNKS, body, 0)
        plsc.subcore_barrier()
        for kk in range(DSLICE // LANES):
            base = s * DSLICE + kk * LANES
            pltpu.sync_copy(cnt.at[pl.ds(base, LANES)], ones_v)
            pltpu.sync_copy(ones_v, out_hbm.at[pl.ds(c * N_DEG + base, LANES)])

    return k(dst_r, ones_rows, zrow)


def _tc_layer(h, agg, deg, Ws, Wn, b, g, be):
    R = 2000

    def body(h_ref, a0, a1, d0, d1, ws, wn, bb, gg, bee, out_ref):
        hh = h_ref[...]
        inv = 1.0 / jnp.maximum(d0[0][:, 0:1] + d1[0][:, 0:1], 1.0)
        mean = (a0[0] + a1[0]) * inv
        y = jnp.dot(hh, ws[...], preferred_element_type=jnp.float32)
        y = y + jnp.dot(mean, wn[...], preferred_element_type=jnp.float32)
        y = y + bb[...] + hh
        y = jnp.maximum(y, 0.0)
        mu = jnp.mean(y, axis=1, keepdims=True)
        xc = y - mu
        var = jnp.mean(xc * xc, axis=1, keepdims=True)
        out_ref[...] = xc / jnp.sqrt(var + 1e-5) * gg[...] + bee[...]

    return pl.pallas_call(
        body,
        grid=(N // R,),
        in_specs=[
            pl.BlockSpec((R, D), lambda i: (i, 0)),
            pl.BlockSpec((1, R, D), lambda i: (0, i, 0)),
            pl.BlockSpec((1, R, D), lambda i: (1, i, 0)),
            pl.BlockSpec((1, R, D), lambda i: (0, i, 0)),
            pl.BlockSpec((1, R, D), lambda i: (1, i, 0)),
            pl.BlockSpec((D, D), lambda i: (0, 0)),
            pl.BlockSpec((D, D), lambda i: (0, 0)),
            pl.BlockSpec((1, D), lambda i: (0, 0)),
            pl.BlockSpec((1, D), lambda i: (0, 0)),
            pl.BlockSpec((1, D), lambda i: (0, 0)),
        ],
        out_specs=pl.BlockSpec((R, D), lambda i: (i, 0)),
        out_shape=jax.ShapeDtypeStruct((N, D), jnp.float32),
    )(h, agg, agg, deg, deg, Ws, Wn, b, g, be)


def _tc_final(h, agg, deg, Ws, Wn, b, Wc, bc):
    R = 2000

    def body(h_ref, a0, a1, d0, d1, ws, wn, bb, wc, bcc, lo_ref, emb_ref):
        hh = h_ref[...]
        inv = 1.0 / jnp.maximum(d0[0][:, 0:1] + d1[0][:, 0:1], 1.0)
        mean = (a0[0] + a1[0]) * inv
        y = jnp.dot(hh, ws[...], preferred_element_type=jnp.float32)
        y = y + jnp.dot(mean, wn[...], preferred_element_type=jnp.float32)
        y = y + bb[...] + hh
        emb_ref[...] = y
        lo_ref[...] = (jnp.dot(y, wc[...], preferred_element_type=jnp.float32)
                       + bcc[...])

    return pl.pallas_call(
        body,
        grid=(N // R,),
        in_specs=[
            pl.BlockSpec((R, D), lambda i: (i, 0)),
            pl.BlockSpec((1, R, D), lambda i: (0, i, 0)),
            pl.BlockSpec((1, R, D), lambda i: (1, i, 0)),
            pl.BlockSpec((1, R, D), lambda i: (0, i, 0)),
            pl.BlockSpec((1, R, D), lambda i: (1, i, 0)),
            pl.BlockSpec((D, D), lambda i: (0, 0)),
            pl.BlockSpec((D, D), lambda i: (0, 0)),
            pl.BlockSpec((1, D), lambda i: (0, 0)),
            pl.BlockSpec((D, C), lambda i: (0, 0)),
            pl.BlockSpec((1, C), lambda i: (0, 0)),
        ],
        out_specs=[
            pl.BlockSpec((R, C), lambda i: (i, 0)),
            pl.BlockSpec((R, D), lambda i: (i, 0)),
        ],
        out_shape=[
            jax.ShapeDtypeStruct((N, C), jnp.float32),
            jax.ShapeDtypeStruct((N, D), jnp.float32),
        ],
    )(h, agg, agg, deg, deg, Ws, Wn, b, Wc, bc)


def kernel(x, edge_index, W_self0, W_neigh0, b0, W_self1, W_neigh1, b1,
           W_self2, W_neigh2, b2, ln_g0, ln_b0, ln_g1, ln_b1, W_cls, b_cls):
    src = edge_index[0]
    dst = edge_index[1]
    pad = E_PAD - E
    src_r = jnp.concatenate(
        [src, jnp.zeros((pad,), jnp.int32)]).reshape(NW * PHASES, PCHUNKS, LANES)
    dst_r = jnp.concatenate(
        [dst, jnp.full((pad,), N, jnp.int32)]).reshape(NW * PHASES, PCHUNKS, LANES)
    zrow = jnp.zeros((LANES, D), jnp.float32)
    ones_rows = jnp.ones((LANES, D), jnp.float32)

    deg = _sc_degree(dst_r, ones_rows, zrow).reshape(NC, N_DEG, D)

    b0r, b1r, b2r = b0.reshape(1, D), b1.reshape(1, D), b2.reshape(1, D)
    g0r, be0r = ln_g0.reshape(1, D), ln_b0.reshape(1, D)
    g1r, be1r = ln_g1.reshape(1, D), ln_b1.reshape(1, D)
    bcr = b_cls.reshape(1, C)

    h = x
    agg = _sc_segment_sum(h, src_r, dst_r, zrow).reshape(NC, N, D)
    h = _tc_layer(h, agg, deg, W_self0, W_neigh0, b0r, g0r, be0r)
    agg = _sc_segment_sum(h, src_r, dst_r, zrow).reshape(NC, N, D)
    h = _tc_layer(h, agg, deg, W_self1, W_neigh1, b1r, g1r, be1r)
    agg = _sc_segment_sum(h, src_r, dst_r, zrow).reshape(NC, N, D)
    logits, emb = _tc_final(h, agg, deg, W_self2, W_neigh2, b2r, W_cls, bcr)
    return (logits, emb)


# speedup vs baseline: 6.8746x; 1.0021x over previous
import functools

import jax
import jax.numpy as jnp
from jax import lax
from jax.experimental import pallas as pl
from jax.experimental.pallas import tpu as pltpu
from jax.experimental.pallas import tpu_sc as plsc

N = 10000
E = 320000
D = 128
C = 64

NC = 2
NS = 16
NW = NC * NS
LANES = 128
CHUNKS = 80
PHASES = 2
PCHUNKS = CHUNKS // PHASES
GSUB = 4
GROWS = LANES // GSUB
EPW = CHUNKS * LANES
E_PAD = NW * EPW
N_AGG = 10240
N_DEG = 10240
DSLICE = N_DEG // NS


def _sc_expand(h, src_r):
    mesh = plsc.VectorSubcoreMesh(core_axis_name="c", subcore_axis_name="s")

    @functools.partial(
        pl.kernel,
        mesh=mesh,
        out_type=jax.ShapeDtypeStruct((E_PAD, D), jnp.float32),
        scratch_types=[
            pltpu.VMEM((PCHUNKS, LANES), jnp.int32),
            pltpu.VMEM((LANES, D), jnp.float32),
            pltpu.VMEM((LANES, D), jnp.float32),
            pltpu.VMEM_SHARED((N, D), jnp.float32),
            pltpu.SemaphoreType.DMA,
            pltpu.SemaphoreType.DMA,
            pltpu.SemaphoreType.DMA,
            pltpu.SemaphoreType.DMA,
        ],
    )
    def k(h_hbm, src_hbm, out_hbm, src_v, buf0, buf1, hst, gsem0, gsem1,
          wsem0, wsem1):
        c = lax.axis_index("c")
        s = lax.axis_index("s")
        wid = c * NS + s
        bufs = (buf0, buf1)
        gsems = (gsem0, gsem1)
        wsems = (wsem0, wsem1)

        full = N // LANES
        for i in range(pl.cdiv(full, NS)):
            m = s + NS * i

            @pl.when(m < full)
            def _():
                pltpu.sync_copy(h_hbm.at[pl.ds(m * LANES, LANES)], buf0)
                pltpu.sync_copy(buf0, hst.at[pl.ds(m * LANES, LANES)])

        tail = N - full * LANES

        @pl.when(s == NS - 1)
        def _():
            pltpu.sync_copy(h_hbm.at[pl.ds(full * LANES, tail)],
                            buf0.at[pl.ds(0, tail)])
            pltpu.sync_copy(buf0.at[pl.ds(0, tail)],
                            hst.at[pl.ds(full * LANES, tail)])

        plsc.subcore_barrier()

        def start_gather(j, b):
            for g in range(GSUB):
                pltpu.async_copy(hst.at[src_v.at[j, pl.ds(g * GROWS, GROWS)]],
                                 bufs[b].at[pl.ds(g * GROWS, GROWS)], gsems[b])

        def wait_gather(b):
            pltpu.make_async_copy(hst.at[src_v.at[0]], bufs[b],
                                  gsems[b]).wait()

        def start_write(base, b):
            pltpu.async_copy(bufs[b], out_hbm.at[pl.ds(base, LANES)], wsems[b])

        def wait_write(b):
            pltpu.make_async_copy(bufs[b], out_hbm.at[pl.ds(0, LANES)],
                                  wsems[b]).wait()

        for p in range(PHASES):
            pltpu.sync_copy(src_hbm.at[wid * PHASES + p], src_v)
            pbase = wid * EPW + p * PCHUNKS * LANES
            start_gather(0, 0)
            start_gather(1, 1)

            def body(j0, carry):
                for b in range(2):
                    j = j0 * 2 + b
                    wait_gather(b)
                    start_write(pbase + j * LANES, b)

                    @pl.when(j + 2 < PCHUNKS)
                    def _():
                        wait_write(b)
                        start_gather(j + 2, b)

                return carry

            lax.fori_loop(0, PCHUNKS // 2, body, 0)
            wait_write(0)
            wait_write(1)

    return k(h, src_r)


def _sc_collect(rows, dst_r, zrow):
    mesh = plsc.VectorSubcoreMesh(core_axis_name="c", subcore_axis_name="s")

    @functools.partial(
        pl.kernel,
        mesh=mesh,
        out_type=jax.ShapeDtypeStruct((NC * N, D), jnp.float32),
        scratch_types=[
            pltpu.VMEM((PCHUNKS, LANES), jnp.int32),
            pltpu.VMEM((LANES, D), jnp.float32),
            pltpu.VMEM((LANES, D), jnp.float32),
            pltpu.VMEM_SHARED((N_AGG, D), jnp.float32),
            pltpu.SemaphoreType.DMA,
            pltpu.SemaphoreType.DMA,
        ],
    )
    def k(rows_hbm, dst_hbm, z_hbm, out_hbm, dst_v, buf0, buf1, acc,
          sem0, sem1):
        c = lax.axis_index("c")
        s = lax.axis_index("s")
        wid = c * NS + s
        bufs = (buf0, buf1)
        sems = (sem0, sem1)

        pltpu.sync_copy(z_hbm, buf0)
        rows0 = N_AGG // NS
        for kk in range(rows0 // LANES):
            pltpu.sync_copy(buf0, acc.at[pl.ds(s * rows0 + kk * LANES, LANES)])
        plsc.subcore_barrier()

        def start_read(base, b):
            pltpu.async_copy(rows_hbm.at[pl.ds(base, LANES)], bufs[b], sems[b])

        def wait_read(b):
            pltpu.make_async_copy(rows_hbm.at[pl.ds(0, LANES)], bufs[b],
                                  sems[b]).wait()

        for p in range(PHASES):
            pltpu.sync_copy(dst_hbm.at[wid * PHASES + p], dst_v)
            pbase = wid * EPW + p * PCHUNKS * LANES
            start_read(pbase, 0)
            start_read(pbase + LANES, 1)

            def body(j0, carry):
                for b in range(2):
                    j = j0 * 2 + b
                    wait_read(b)
                    pltpu.sync_copy(bufs[b], acc.at[dst_v.at[j]], add=True)

                    @pl.when(j + 2 < PCHUNKS)
                    def _():
                        start_read(pbase + (j + 2) * LANES, b)

                return carry

            lax.fori_loop(0, PCHUNKS // 2, body, 0)
        plsc.subcore_barrier()

        full = N // LANES
        for i in range(pl.cdiv(full, NS)):
            m = s + NS * i

            @pl.when(m < full)
            def _():
                pltpu.sync_copy(acc.at[pl.ds(m * LANES, LANES)], buf0)
                pltpu.sync_copy(buf0, out_hbm.at[pl.ds(c * N + m * LANES, LANES)])

        tail = N - full * LANES

        @pl.when(s == NS - 1)
        def _():
            pltpu.sync_copy(acc.at[pl.ds(full * LANES, tail)],
                            buf0.at[pl.ds(0, tail)])
            pltpu.sync_copy(buf0.at[pl.ds(0, tail)],
                            out_hbm.at[pl.ds(c * N + full * LANES, tail)])

    return k(rows, dst_r, zrow)


def _sc_segment_sum(h, src_r, dst_r, zrow):
    rows = _sc_expand(h, src_r)
    return _sc_collect(rows, dst_r, zrow)


def _sc_degree(dst_r, ones_rows, zrow):
    mesh = plsc.VectorSubcoreMesh(core_axis_name="c", subcore_axis_name="s")

    @functools.partial(
        pl.kernel,
        mesh=mesh,
        out_type=jax.ShapeDtypeStruct((NC * N_DEG, D), jnp.float32),
        scratch_types=[
            pltpu.VMEM((PCHUNKS, LANES), jnp.int32),
            pltpu.VMEM((LANES, D), jnp.float32),
            pltpu.VMEM_SHARED((N_DEG, D), jnp.float32),
        ],
    )
    def k(dst_hbm, ones_hbm, z_hbm, out_hbm, idx_v, ones_v, cnt):
        c = lax.axis_index("c")
        s = lax.axis_index("s")
        wid = c * NS + s
        pltpu.sync_copy(z_hbm, ones_v)
        for kk in range(DSLICE // LANES):
            pltpu.sync_copy(ones_v, cnt.at[pl.ds(s * DSLICE + kk * LANES, LANES)])
        pltpu.sync_copy(ones_hbm, ones_v)
        plsc.subcore_barrier()

        def body(j, carry):
            pltpu.sync_copy(ones_v, cnt.at[idx_v.at[j]], add=True)
            return carry

        for p in range(PHASES):
            pltpu.sync_copy(dst_hbm.at[wid * PHASES + p], idx_v)
            lax.fori_loop(0, PCHUNKS, body, 0)
        plsc.subcore_barrier()
        for kk in range(DSLICE // LANES):
            base = s * DSLICE + kk * LANES
            pltpu.sync_copy(cnt.at[pl.ds(base, LANES)], ones_v)
            pltpu.sync_copy(ones_v, out_hbm.at[pl.ds(c * N_DEG + base, LANES)])

    return k(dst_r, ones_rows, zrow)


def _tc_invdeg(deg):
    R = 2000

    def body(d0, d1, out_ref):
        out_ref[...] = 1.0 / jnp.maximum(d0[0][:, 0:1] + d1[0][:, 0:1], 1.0)

    return pl.pallas_call(
        body,
        grid=(N // R,),
        in_specs=[
            pl.BlockSpec((1, R, D), lambda i: (0, i, 0)),
            pl.BlockSpec((1, R, D), lambda i: (1, i, 0)),
        ],
        out_specs=pl.BlockSpec((R, 1), lambda i: (i, 0)),
        out_shape=jax.ShapeDtypeStruct((N, 1), jnp.float32),
    )(deg, deg)


def _tc_layer(h, agg, inv, Ws, Wn, b, g, be):
    R = 2000

    def body(h_ref, a0, a1, inv_ref, ws, wn, bb, gg, bee, out_ref):
        hh = h_ref[...]
        mean = (a0[0] + a1[0]) * inv_ref[...]
        y = jnp.dot(hh, ws[...], preferred_element_type=jnp.float32)
        y = y + jnp.dot(mean, wn[...], preferred_element_type=jnp.float32)
        y = y + bb[...] + hh
        y = jnp.maximum(y, 0.0)
        mu = jnp.mean(y, axis=1, keepdims=True)
        xc = y - mu
        var = jnp.mean(xc * xc, axis=1, keepdims=True)
        out_ref[...] = xc / jnp.sqrt(var + 1e-5) * gg[...] + bee[...]

    return pl.pallas_call(
        body,
        grid=(N // R,),
        in_specs=[
            pl.BlockSpec((R, D), lambda i: (i, 0)),
            pl.BlockSpec((1, R, D), lambda i: (0, i, 0)),
            pl.BlockSpec((1, R, D), lambda i: (1, i, 0)),
            pl.BlockSpec((R, 1), lambda i: (i, 0)),
            pl.BlockSpec((D, D), lambda i: (0, 0)),
            pl.BlockSpec((D, D), lambda i: (0, 0)),
            pl.BlockSpec((1, D), lambda i: (0, 0)),
            pl.BlockSpec((1, D), lambda i: (0, 0)),
            pl.BlockSpec((1, D), lambda i: (0, 0)),
        ],
        out_specs=pl.BlockSpec((R, D), lambda i: (i, 0)),
        out_shape=jax.ShapeDtypeStruct((N, D), jnp.float32),
    )(h, agg, agg, inv, Ws, Wn, b, g, be)


def _tc_final(h, agg, inv, Ws, Wn, b, Wc, bc):
    R = 2000

    def body(h_ref, a0, a1, inv_ref, ws, wn, bb, wc, bcc, lo_ref, emb_ref):
        hh = h_ref[...]
        mean = (a0[0] + a1[0]) * inv_ref[...]
        y = jnp.dot(hh, ws[...], preferred_element_type=jnp.float32)
        y = y + jnp.dot(mean, wn[...], preferred_element_type=jnp.float32)
        y = y + bb[...] + hh
        emb_ref[...] = y
        lo_ref[...] = (jnp.dot(y, wc[...], preferred_element_type=jnp.float32)
                       + bcc[...])

    return pl.pallas_call(
        body,
        grid=(N // R,),
        in_specs=[
            pl.BlockSpec((R, D), lambda i: (i, 0)),
            pl.BlockSpec((1, R, D), lambda i: (0, i, 0)),
            pl.BlockSpec((1, R, D), lambda i: (1, i, 0)),
            pl.BlockSpec((R, 1), lambda i: (i, 0)),
            pl.BlockSpec((D, D), lambda i: (0, 0)),
            pl.BlockSpec((D, D), lambda i: (0, 0)),
            pl.BlockSpec((1, D), lambda i: (0, 0)),
            pl.BlockSpec((D, C), lambda i: (0, 0)),
            pl.BlockSpec((1, C), lambda i: (0, 0)),
        ],
        out_specs=[
            pl.BlockSpec((R, C), lambda i: (i, 0)),
            pl.BlockSpec((R, D), lambda i: (i, 0)),
        ],
        out_shape=[
            jax.ShapeDtypeStruct((N, C), jnp.float32),
            jax.ShapeDtypeStruct((N, D), jnp.float32),
        ],
    )(h, agg, agg, inv, Ws, Wn, b, Wc, bc)


def kernel(x, edge_index, W_self0, W_neigh0, b0, W_self1, W_neigh1, b1,
           W_self2, W_neigh2, b2, ln_g0, ln_b0, ln_g1, ln_b1, W_cls, b_cls):
    src = edge_index[0]
    dst = edge_index[1]
    pad = E_PAD - E
    src_r = jnp.concatenate(
        [src, jnp.zeros((pad,), jnp.int32)]).reshape(NW * PHASES, PCHUNKS, LANES)
    dst_r = jnp.concatenate(
        [dst, jnp.full((pad,), N, jnp.int32)]).reshape(NW * PHASES, PCHUNKS, LANES)
    zrow = jnp.zeros((LANES, D), jnp.float32)
    ones_rows = jnp.ones((LANES, D), jnp.float32)

    deg = _sc_degree(dst_r, ones_rows, zrow).reshape(NC, N_DEG, D)
    inv = _tc_invdeg(deg)

    b0r, b1r, b2r = b0.reshape(1, D), b1.reshape(1, D), b2.reshape(1, D)
    g0r, be0r = ln_g0.reshape(1, D), ln_b0.reshape(1, D)
    g1r, be1r = ln_g1.reshape(1, D), ln_b1.reshape(1, D)
    bcr = b_cls.reshape(1, C)

    h = x
    agg = _sc_segment_sum(h, src_r, dst_r, zrow).reshape(NC, N, D)
    h = _tc_layer(h, agg, inv, W_self0, W_neigh0, b0r, g0r, be0r)
    agg = _sc_segment_sum(h, src_r, dst_r, zrow).reshape(NC, N, D)
    h = _tc_layer(h, agg, inv, W_self1, W_neigh1, b1r, g1r, be1r)
    agg = _sc_segment_sum(h, src_r, dst_r, zrow).reshape(NC, N, D)
    logits, emb = _tc_final(h, agg, inv, W_self2, W_neigh2, b2r, W_cls, bcr)
    return (logits, emb)
